# baseline (speedup 1.0000x reference)
import functools

import jax
import jax.numpy as jnp
from jax import lax
from jax.experimental import pallas as pl
from jax.experimental.pallas import tpu as pltpu
from jax.experimental.pallas import tpu_sc as plsc

VOCAB = 1000000
EMBED_DIM = 32
BATCH = 16384
HIST = 50

NC = 2
NS = 16
NW = NC * NS

B_FLAT = BATCH * HIST
B_PER_W = B_FLAT // NW
CHUNK = 128
N_CHUNKS = B_PER_W // CHUNK
SUPER = 10
SUPER_ROWS = SUPER * CHUNK
N_SUPER = N_CHUNKS // SUPER

TCOLS = (VOCAB + 127) // 128
STEPS = (TCOLS + NW - 1) // NW
STEPS_PAD = STEPS + (STEPS % 2)
LAST_START = (TCOLS - 1) * 128
SCRATCH = (LAST_START + 128) * EMBED_DIM


def _make_transpose():
  mesh = plsc.VectorSubcoreMesh(core_axis_name="c", subcore_axis_name="s")

  @functools.partial(
      pl.kernel,
      out_type=jax.ShapeDtypeStruct((SCRATCH,), jnp.float32),
      mesh=mesh,
      scratch_types=[
          pltpu.VMEM((EMBED_DIM, 128), jnp.float32),
          pltpu.VMEM((EMBED_DIM, 128), jnp.float32),
          pltpu.VMEM((128 * EMBED_DIM,), jnp.float32),
          pltpu.VMEM((128 * EMBED_DIM,), jnp.float32),
          pltpu.SemaphoreType.DMA,
          pltpu.SemaphoreType.DMA,
          pltpu.SemaphoreType.DMA,
          pltpu.SemaphoreType.DMA,
      ],
      compiler_params=pltpu.CompilerParams(
          use_tc_tiling_on_sc=True, needs_layout_passes=False),
  )
  def transpose_kernel(embt_hbm, scratch_hbm, tb0, tb1, ob0, ob1,
                       isem0, isem1, osem0, osem1):
    wid = lax.axis_index("s") * NC + lax.axis_index("c")
    iota_d = lax.iota(jnp.int32, 16) * EMBED_DIM

    def col_start(step):
      return pl.multiple_of(
          jnp.minimum((step * NW + wid) * 128, LAST_START), 128)

    def valid(step):
      return (step * NW + wid) < TCOLS

    def fire_in(step, tb, isem):
      pltpu.async_copy(
          embt_hbm.at[:, pl.ds(col_start(step), 128)], tb, isem)

    def wait_in(step, tb, isem):
      pltpu.make_async_copy(
          embt_hbm.at[:, pl.ds(col_start(step), 128)], tb, isem).wait()

    def fire_out(step, ob, osem):
      pltpu.async_copy(
          ob, scratch_hbm.at[pl.ds(col_start(step) * EMBED_DIM,
                                   128 * EMBED_DIM)], osem)

    def wait_out(step, ob, osem):
      pltpu.make_async_copy(
          ob, scratch_hbm.at[pl.ds(col_start(step) * EMBED_DIM,
                                   128 * EMBED_DIM)], osem).wait()

    def compute(tb, ob):
      for d in range(EMBED_DIM):
        for r0 in range(0, 128, 16):
          v = tb[d, pl.ds(r0, 16)]
          plsc.store_scatter(ob, [iota_d + (r0 * EMBED_DIM + d)], v)

    bufs = ((tb0, ob0, isem0, osem0), (tb1, ob1, isem1, osem1))

    @pl.when(valid(0))
    def _():
      fire_in(0, tb0, isem0)

    @pl.when(valid(1))
    def _():
      fire_in(1, tb1, isem1)

    def body(i, carry):
      for b in range(2):
        tb, ob, isem, osem = bufs[b]
        step = 2 * i + b

        @pl.when(valid(step))
        def _():
          wait_in(step, tb, isem)

          @pl.when(step >= 2)
          def _():
            wait_out(step - 2, ob, osem)

          compute(tb, ob)
          fire_out(step, ob, osem)

          @pl.when(valid(step + 2))
          def _():
            fire_in(step + 2, tb, isem)
      return carry

    lax.fori_loop(0, STEPS_PAD // 2, body, 0)
    last0 = STEPS_PAD - 2
    last1 = STEPS_PAD - 1

    @pl.when(valid(last0))
    def _():
      wait_out(last0, ob0, osem0)

    @pl.when(valid(last1))
    def _():
      wait_out(last1, ob1, osem1)

  return transpose_kernel


def _make_gather():
  mesh = plsc.VectorSubcoreMesh(core_axis_name="c", subcore_axis_name="s")

  @functools.partial(
      pl.kernel,
      out_type=jax.ShapeDtypeStruct((NW, B_PER_W, EMBED_DIM), jnp.float32),
      mesh=mesh,
      scratch_types=[
          pltpu.VMEM((N_CHUNKS, CHUNK), jnp.int32),
          pltpu.VMEM((SUPER_ROWS, EMBED_DIM), jnp.float32),
          pltpu.VMEM((SUPER_ROWS, EMBED_DIM), jnp.float32),
          pltpu.SemaphoreType.DMA,
          pltpu.SemaphoreType.DMA,
          pltpu.SemaphoreType.DMA,
          pltpu.SemaphoreType.DMA,
      ],
      compiler_params=pltpu.CompilerParams(use_tc_tiling_on_sc=False),
  )
  def gather_kernel(table_hbm, idx_hbm, out_hbm, idx_v, rows0, rows1,
                    gsem0, gsem1, osem0, osem1):
    wid = lax.axis_index("s") * NC + lax.axis_index("c")
    out_w = out_hbm.at[wid]
    pltpu.sync_copy(idx_hbm.at[wid], idx_v)

    def fire_gathers(s, rows, gsem):
      for j in range(SUPER):
        pltpu.async_copy(
            table_hbm.at[idx_v.at[s * SUPER + j]],
            rows.at[pl.ds(j * CHUNK, CHUNK)],
            gsem,
        )

    def drain_and_writeback(s, rows, gsem, osem):
      pltpu.make_async_copy(table_hbm.at[pl.ds(0, SUPER_ROWS)], rows,
                            gsem).wait()
      pltpu.async_copy(rows, out_w.at[pl.ds(s * SUPER_ROWS, SUPER_ROWS)], osem)

    def wait_writeback(s, rows, osem):
      pltpu.make_async_copy(
          rows, out_w.at[pl.ds(s * SUPER_ROWS, SUPER_ROWS)], osem).wait()

    bufs = ((rows0, gsem0, osem0), (rows1, gsem1, osem1))

    def body(i, carry):
      for b in range(2):
        rows, gsem, osem = bufs[b]
        prows, pgsem, posem = bufs[1 - b]
        s = 2 * i + b

        @pl.when(i >= 1)
        def _():
          wait_writeback(s - 2, rows, osem)

        fire_gathers(s, rows, gsem)

        if b == 1:
          drain_and_writeback(s - 1, prows, pgsem, posem)
        else:
          @pl.when(i >= 1)
          def _():
            drain_and_writeback(s - 1, prows, pgsem, posem)
      return carry

    lax.fori_loop(0, N_SUPER // 2, body, 0)
    drain_and_writeback(N_SUPER - 1, rows1, gsem1, osem1)
    wait_writeback(N_SUPER - 2, rows0, osem0)
    wait_writeback(N_SUPER - 1, rows1, osem1)

  return gather_kernel


_transpose = _make_transpose()
_gather = _make_gather()

@jax.jit
def kernel(inputs, embeddings):
  idx = inputs.astype(jnp.int32).reshape(NW, N_CHUNKS, CHUNK)
  table = _transpose(embeddings.T).reshape(
      SCRATCH // EMBED_DIM, EMBED_DIM)
  out = _gather(table, idx)
  return out.reshape(BATCH, HIST, EMBED_DIM)


# speedup vs baseline: 1.2085x; 1.2085x over previous
import functools

import jax
import jax.numpy as jnp
from jax import lax
from jax.experimental import pallas as pl
from jax.experimental.pallas import tpu as pltpu
from jax.experimental.pallas import tpu_sc as plsc

VOCAB = 1000000
EMBED_DIM = 32
BATCH = 16384
HIST = 50

NC = 2
NS = 16
NW = NC * NS

B_FLAT = BATCH * HIST
B_PER_W = B_FLAT // NW
CHUNK = 128
N_CHUNKS = B_PER_W // CHUNK
SUPER = 10
SUPER_ROWS = SUPER * CHUNK
N_SUPER = N_CHUNKS // SUPER

TCOLS = (VOCAB + 127) // 128
STEPS = (TCOLS + NW - 1) // NW
STEPS_PAD = STEPS + (STEPS % 2)
LAST_START = (TCOLS - 1) * 128
SCRATCH = (LAST_START + 128) * EMBED_DIM
OBW = 33


def _make_transpose():
  mesh = plsc.VectorSubcoreMesh(core_axis_name="c", subcore_axis_name="s")

  @functools.partial(
      pl.kernel,
      out_type=jax.ShapeDtypeStruct((SCRATCH,), jnp.float32),
      mesh=mesh,
      scratch_types=[
          pltpu.VMEM((EMBED_DIM, 128), jnp.float32),
          pltpu.VMEM((EMBED_DIM, 128), jnp.float32),
          pltpu.VMEM((128 * OBW,), jnp.float32),
          pltpu.VMEM((128 * EMBED_DIM,), jnp.float32),
          pltpu.VMEM((128 * EMBED_DIM,), jnp.float32),
          pltpu.SemaphoreType.DMA,
          pltpu.SemaphoreType.DMA,
          pltpu.SemaphoreType.DMA,
          pltpu.SemaphoreType.DMA,
      ],
      compiler_params=pltpu.CompilerParams(
          use_tc_tiling_on_sc=True, needs_layout_passes=False),
  )
  def transpose_kernel(embt_hbm, scratch_hbm, tb0, tb1, ob, cb0, cb1,
                       isem0, isem1, osem0, osem1):
    wid = lax.axis_index("s") * NC + lax.axis_index("c")
    iota33 = lax.iota(jnp.int32, 16) * OBW

    def col_start(step):
      return pl.multiple_of(
          jnp.minimum((step * NW + wid) * 128, LAST_START), 128)

    def valid(step):
      return (step * NW + wid) < TCOLS

    def fire_in(step, tb, isem):
      pltpu.async_copy(
          embt_hbm.at[:, pl.ds(col_start(step), 128)], tb, isem)

    def wait_in(step, tb, isem):
      pltpu.make_async_copy(
          embt_hbm.at[:, pl.ds(col_start(step), 128)], tb, isem).wait()

    def fire_out(step, cb, osem):
      pltpu.async_copy(
          cb, scratch_hbm.at[pl.ds(col_start(step) * EMBED_DIM,
                                   128 * EMBED_DIM)], osem)

    def wait_out(step, cb, osem):
      pltpu.make_async_copy(
          cb, scratch_hbm.at[pl.ds(col_start(step) * EMBED_DIM,
                                   128 * EMBED_DIM)], osem).wait()

    def compute(tb, cb):
      for d in range(EMBED_DIM):
        for r0 in range(0, 128, 16):
          v = tb[d, pl.ds(r0, 16)]
          plsc.store_scatter(ob, [iota33 + (r0 * OBW + d)], v)
      for r in range(128):
        cb[pl.ds(r * EMBED_DIM, 16)] = ob[pl.ds(r * OBW, 16)]
        cb[pl.ds(r * EMBED_DIM + 16, 16)] = ob[pl.ds(r * OBW + 16, 16)]

    bufs = ((tb0, cb0, isem0, osem0), (tb1, cb1, isem1, osem1))

    @pl.when(valid(0))
    def _():
      fire_in(0, tb0, isem0)

    @pl.when(valid(1))
    def _():
      fire_in(1, tb1, isem1)

    def body(i, carry):
      for b in range(2):
        tb, cb, isem, osem = bufs[b]
        step = 2 * i + b

        @pl.when(valid(step))
        def _():
          wait_in(step, tb, isem)

          @pl.when(step >= 2)
          def _():
            wait_out(step - 2, cb, osem)

          compute(tb, cb)
          fire_out(step, cb, osem)

          @pl.when(valid(step + 2))
          def _():
            fire_in(step + 2, tb, isem)
      return carry

    lax.fori_loop(0, STEPS_PAD // 2, body, 0)
    last0 = STEPS_PAD - 2
    last1 = STEPS_PAD - 1

    @pl.when(valid(last0))
    def _():
      wait_out(last0, cb0, osem0)

    @pl.when(valid(last1))
    def _():
      wait_out(last1, cb1, osem1)

  return transpose_kernel


def _make_gather():
  mesh = plsc.VectorSubcoreMesh(core_axis_name="c", subcore_axis_name="s")

  @functools.partial(
      pl.kernel,
      out_type=jax.ShapeDtypeStruct((NW, B_PER_W, EMBED_DIM), jnp.float32),
      mesh=mesh,
      scratch_types=[
          pltpu.VMEM((N_CHUNKS, CHUNK), jnp.int32),
          pltpu.VMEM((SUPER_ROWS, EMBED_DIM), jnp.float32),
          pltpu.VMEM((SUPER_ROWS, EMBED_DIM), jnp.float32),
          pltpu.SemaphoreType.DMA,
          pltpu.SemaphoreType.DMA,
          pltpu.SemaphoreType.DMA,
          pltpu.SemaphoreType.DMA,
      ],
      compiler_params=pltpu.CompilerParams(use_tc_tiling_on_sc=False),
  )
  def gather_kernel(table_hbm, idx_hbm, out_hbm, idx_v, rows0, rows1,
                    gsem0, gsem1, osem0, osem1):
    wid = lax.axis_index("s") * NC + lax.axis_index("c")
    out_w = out_hbm.at[wid]
    pltpu.sync_copy(idx_hbm.at[wid], idx_v)

    def fire_gathers(s, rows, gsem):
      for j in range(SUPER):
        pltpu.async_copy(
            table_hbm.at[idx_v.at[s * SUPER + j]],
            rows.at[pl.ds(j * CHUNK, CHUNK)],
            gsem,
        )

    def drain_and_writeback(s, rows, gsem, osem):
      pltpu.make_async_copy(table_hbm.at[pl.ds(0, SUPER_ROWS)], rows,
                            gsem).wait()
      pltpu.async_copy(rows, out_w.at[pl.ds(s * SUPER_ROWS, SUPER_ROWS)], osem)

    def wait_writeback(s, rows, osem):
      pltpu.make_async_copy(
          rows, out_w.at[pl.ds(s * SUPER_ROWS, SUPER_ROWS)], osem).wait()

    bufs = ((rows0, gsem0, osem0), (rows1, gsem1, osem1))

    def body(i, carry):
      for b in range(2):
        rows, gsem, osem = bufs[b]
        prows, pgsem, posem = bufs[1 - b]
        s = 2 * i + b

        @pl.when(i >= 1)
        def _():
          wait_writeback(s - 2, rows, osem)

        fire_gathers(s, rows, gsem)

        if b == 1:
          drain_and_writeback(s - 1, prows, pgsem, posem)
        else:
          @pl.when(i >= 1)
          def _():
            drain_and_writeback(s - 1, prows, pgsem, posem)
      return carry

    lax.fori_loop(0, N_SUPER // 2, body, 0)
    drain_and_writeback(N_SUPER - 1, rows1, gsem1, osem1)
    wait_writeback(N_SUPER - 2, rows0, osem0)
    wait_writeback(N_SUPER - 1, rows1, osem1)

  return gather_kernel


_transpose = _make_transpose()
_gather = _make_gather()

@jax.jit
def kernel(inputs, embeddings):
  idx = inputs.astype(jnp.int32).reshape(NW, N_CHUNKS, CHUNK)
  table = _transpose(embeddings.T).reshape(
      SCRATCH // EMBED_DIM, EMBED_DIM)
  out = _gather(table, idx)
  return out.reshape(BATCH, HIST, EMBED_DIM)
